# Initial kernel scaffold; baseline (speedup 1.0000x reference)
import jax
import jax.numpy as jnp
from jax.experimental import pallas as pl


def kernel(x, edge_index, batch, W_l0, b_l0, W_r0, W_l1, b_l1, W_r1, W_l2, b_l2, W_r2, prelu_a):
    raise NotImplementedError("write your pallas kernel here")


# speedup vs baseline: 3.4546x; 3.4546x over previous
import functools

import jax
import jax.numpy as jnp
from jax import lax
from jax.experimental import pallas as pl
from jax.experimental.pallas import tpu as pltpu
from jax.experimental.pallas import tpu_sc as plsc

N = 10000
E = 160000
D_IN = 256
D_H = 512
NG = 16

NT = 16
NPAD = 10240
RPT = NPAD // NT
K = 128
STEPS = 79
EPT = K * STEPS
EPAD = NT * EPT
RB = 256
GRID = NPAD // RB


def _make_sc_agg(n_chunks):
    cps = n_chunks // 2
    mesh = plsc.VectorSubcoreMesh(core_axis_name="c", subcore_axis_name="s",
                                  num_cores=2, num_subcores=NT)
    out_type = [jax.ShapeDtypeStruct((n_chunks, NPAD, 128), jnp.float32)]
    scratch = [
        pltpu.VMEM((STEPS, K), jnp.int32),
        pltpu.VMEM((STEPS, K), jnp.int32),
        pltpu.VMEM((K, 128), jnp.float32),
        pltpu.VMEM_SHARED((NPAD, 128), jnp.float32),
        pltpu.SemaphoreType.DMA,
    ]

    def body(y, srcr, dstr, z128, out, idx_s, idx_d, rows, aggsh, sem):
        c = lax.axis_index("c")
        s = lax.axis_index("s")
        r0 = s * RPT
        pltpu.sync_copy(srcr.at[s], idx_s)
        pltpu.sync_copy(dstr.at[s], idx_d)
        for qi in range(cps):
            q = c * cps + qi
            pltpu.sync_copy(z128.at[pl.ds(r0, RPT)], aggsh.at[pl.ds(r0, RPT)])
            plsc.subcore_barrier()

            def step(j, carry):
                pltpu.async_copy(y.at[q].at[idx_s.at[j]], rows, sem).wait()
                pltpu.sync_copy(rows, aggsh.at[idx_d.at[j]], add=True)
                return carry
            lax.fori_loop(0, STEPS, step, 0)
            plsc.subcore_barrier()
            pltpu.sync_copy(aggsh.at[pl.ds(r0, RPT)],
                            out.at[q, pl.ds(r0, RPT)])

    return pl.kernel(body, out_type=out_type, mesh=mesh,
                     scratch_types=scratch)


def _make_sc_cnt():
    mesh = plsc.VectorSubcoreMesh(core_axis_name="c", subcore_axis_name="s",
                                  num_cores=2, num_subcores=NT)
    out_type = [jax.ShapeDtypeStruct((NPAD, 128), jnp.float32)]
    scratch = [
        pltpu.VMEM((STEPS, K), jnp.int32),
        pltpu.VMEM((K, 128), jnp.float32),
        pltpu.VMEM_SHARED((NPAD, 128), jnp.float32),
        pltpu.SemaphoreType.DMA,
    ]

    def body(dstr, z128, ones_h, cnt_out, idx_d, ones_v, cntsh, sem):
        c = lax.axis_index("c")
        s = lax.axis_index("s")
        r0 = s * RPT

        @pl.when(c == 0)
        def _():
            pltpu.sync_copy(dstr.at[s], idx_d)
            pltpu.sync_copy(ones_h, ones_v)
            pltpu.sync_copy(z128.at[pl.ds(r0, RPT)],
                            cntsh.at[pl.ds(r0, RPT)])
        plsc.subcore_barrier()

        @pl.when(c == 0)
        def _():
            def step(j, carry):
                pltpu.sync_copy(ones_v, cntsh.at[idx_d.at[j]], add=True)
                return carry
            lax.fori_loop(0, STEPS, step, 0)
        plsc.subcore_barrier()

        @pl.when(c == 0)
        def _():
            pltpu.sync_copy(cntsh.at[pl.ds(r0, RPT)],
                            cnt_out.at[pl.ds(r0, RPT)])

    return pl.kernel(body, out_type=out_type, mesh=mesh,
                     scratch_types=scratch)


_SC_AGG2 = _make_sc_agg(2)
_SC_AGG4 = _make_sc_agg(4)
_SC_CNT = _make_sc_cnt()


def _make_tc_layer(din, last):
    nin = din // 128

    def body(*refs):
        if last:
            (agg_ref, cnt_ref, h_ref, wl_ref, b_ref, wr_ref, bt_ref,
             h_out, g_out) = refs
        else:
            (agg_ref, cnt_ref, h_ref, wl_ref, b_ref, wr_ref, a_ref,
             h_out, hch_out) = refs
        aggf = jnp.concatenate([agg_ref[qq] for qq in range(nin)], axis=1)
        cnt = cnt_ref[:, 0:1]
        mean = aggf * (1.0 / jnp.maximum(cnt, 1.0))
        out = (jnp.dot(mean, wl_ref[...], preferred_element_type=jnp.float32)
               + b_ref[...]
               + jnp.dot(h_ref[...], wr_ref[...],
                         preferred_element_type=jnp.float32))
        if not last:
            a = a_ref[...]
            out = jnp.where(out >= 0.0, out, a * out)
            h_out[...] = out
            for qq in range(4):
                hch_out[qq] = out[:, qq * 128:(qq + 1) * 128]
        else:
            h_out[...] = out
            i = pl.program_id(0)

            @pl.when(i == 0)
            def _():
                g_out[...] = jnp.full((NG, D_H), -jnp.inf, jnp.float32)
            bt = bt_ref[...]
            loc = jnp.stack(
                [jnp.max(jnp.where(bt == gg, out, -jnp.inf), axis=0)
                 for gg in range(NG)], axis=0)
            g_out[...] = jnp.maximum(g_out[...], loc)

    in_specs = [
        pl.BlockSpec((nin, RB, 128), lambda i: (0, i, 0)),
        pl.BlockSpec((RB, 128), lambda i: (i, 0)),
        pl.BlockSpec((RB, din), lambda i: (i, 0)),
        pl.BlockSpec((din, D_H), lambda i: (0, 0)),
        pl.BlockSpec((1, D_H), lambda i: (0, 0)),
        pl.BlockSpec((din, D_H), lambda i: (0, 0)),
    ]
    if last:
        in_specs.append(pl.BlockSpec((RB, 1), lambda i: (i, 0)))
        out_specs = [
            pl.BlockSpec((RB, D_H), lambda i: (i, 0)),
            pl.BlockSpec((NG, D_H), lambda i: (0, 0)),
        ]
        out_shape = [
            jax.ShapeDtypeStruct((NPAD, D_H), jnp.float32),
            jax.ShapeDtypeStruct((NG, D_H), jnp.float32),
        ]
    else:
        in_specs.append(pl.BlockSpec((1, D_H), lambda i: (0, 0)))
        out_specs = [
            pl.BlockSpec((RB, D_H), lambda i: (i, 0)),
            pl.BlockSpec((4, RB, 128), lambda i: (0, i, 0)),
        ]
        out_shape = [
            jax.ShapeDtypeStruct((NPAD, D_H), jnp.float32),
            jax.ShapeDtypeStruct((4, NPAD, 128), jnp.float32),
        ]

    return pl.pallas_call(
        body,
        grid=(GRID,),
        in_specs=in_specs,
        out_specs=out_specs,
        out_shape=out_shape,
    )


_TC_L0 = _make_tc_layer(D_IN, last=False)
_TC_L1 = _make_tc_layer(D_H, last=False)
_TC_L2 = _make_tc_layer(D_H, last=True)


@jax.jit
def kernel(x, edge_index, batch, W_l0, b_l0, W_r0, W_l1, b_l1, W_r1,
           W_l2, b_l2, W_r2, prelu_a):
    src, dst = edge_index[0], edge_index[1]
    pad_e = EPAD - E
    src_r = jnp.concatenate(
        [src, jnp.zeros((pad_e,), jnp.int32)]).reshape(NT, STEPS, K)
    dst_r = jnp.concatenate(
        [dst, jnp.full((pad_e,), NPAD - 1, jnp.int32)]).reshape(NT, STEPS, K)

    x_pad = jnp.pad(x, ((0, NPAD - N), (0, 0)))
    x_ch = x_pad.reshape(NPAD, D_IN // 128, 128).transpose(1, 0, 2)
    z128 = jnp.zeros((NPAD, 128), jnp.float32)
    ones128 = jnp.ones((K, 128), jnp.float32)
    batch_p = jnp.pad(batch, (0, NPAD - N),
                      constant_values=NG).reshape(NPAD, 1)

    bl0 = b_l0.reshape(1, D_H)
    bl1 = b_l1.reshape(1, D_H)
    bl2 = b_l2.reshape(1, D_H)
    a_r = prelu_a.reshape(1, D_H)

    (cnt16,) = _SC_CNT(dst_r, z128, ones128)
    (agg0,) = _SC_AGG2(x_ch, src_r, dst_r, z128)
    h1, h1ch = _TC_L0(agg0, cnt16, x_pad, W_l0, bl0, W_r0, a_r)
    (agg1,) = _SC_AGG4(h1ch, src_r, dst_r, z128)
    h2, h2ch = _TC_L1(agg1, cnt16, h1, W_l1, bl1, W_r1, a_r)
    (agg2,) = _SC_AGG4(h2ch, src_r, dst_r, z128)
    h3, g = _TC_L2(agg2, cnt16, h2, W_l2, bl2, W_r2, batch_p)
    return h3[:N], g
